# Initial kernel scaffold; baseline (speedup 1.0000x reference)
import jax
import jax.numpy as jnp
from jax.experimental import pallas as pl


def kernel(node, edge, edge_index, W_lin, b_lin, W_att, b_att, W_alpha):
    raise NotImplementedError("write your pallas kernel here")


# speedup vs baseline: 5.1588x; 5.1588x over previous
import functools

import jax
import jax.numpy as jnp
from jax import lax
from jax.experimental import pallas as pl
from jax.experimental.pallas import tpu as pltpu
from jax.experimental.pallas import tpu_sc as plsc

NC = 2
NS = 16
L = 16
NW = NC * NS


def _leaky(x):
    return jnp.where(x > 0, x, 0.2 * x)


def _pre_body(U, UW, H, node_ref, wcat_ref, bl_ref, ba_ref,
              wn_ref, pin_ref, pout_ref):
    x = node_ref[...]
    HU = H * U
    p = jnp.dot(x, wcat_ref[...], preferred_element_type=jnp.float32)
    wn = _leaky(p[:, :HU] + bl_ref[...])
    pin_ref[...] = p[:, HU:2 * HU] + ba_ref[...]
    pout_ref[...] = p[:, 2 * HU:]
    rb = x.shape[0]
    pad = (lax.broadcasted_iota(jnp.int32, (rb, UW - U), 1) == 0)
    for k in range(H):
        wn_ref[k, :, :U] = wn[:, k * U:(k + 1) * U]
        wn_ref[k, :, U:] = pad.astype(jnp.float32)


def _score_body(E, EW, BA, H, U, UW,
                pin_hbm, pout_hbm, ii_hbm, io_hbm, al_hbm, a_hbm, e_hbm,
                ii_v, io_v, pin_v, pout_v, a_v, e_v, al_v, sem1, sem2):
    wid = lax.axis_index("s") * NC + lax.axis_index("c")
    base = wid * EW
    pltpu.sync_copy(al_hbm, al_v)

    def blk_body(b, _):
        off = base + b * BA
        pltpu.sync_copy(ii_hbm.at[pl.ds(off, BA)], ii_v)
        pltpu.sync_copy(io_hbm.at[pl.ds(off, BA)], io_v)
        cp1 = pltpu.async_copy(pin_hbm.at[ii_v], pin_v, sem1)
        cp2 = pltpu.async_copy(pout_hbm.at[io_v], pout_v, sem2)
        cp1.wait()
        cp2.wait()

        epg = L // H
        lanes = lax.iota(jnp.int32, L)
        tr_idx0 = (lanes % H) * BA + lanes // H

        def group_body(g, _):
            vec = jnp.zeros((L,), jnp.float32)
            for eo in range(epg):
                e = g * epg + eo
                for k in range(H):
                    acc = jnp.zeros((L,), jnp.float32)
                    for c in range(U // L):
                        sl = pl.ds(k * U + c * L, L)
                        x = pin_v[e, sl] + pout_v[e, sl]
                        acc = acc + _leaky(x) * al_v[k, pl.ds(c * L, L)]
                    vec = jnp.where(lanes == eo * H + k, jnp.sum(acc), vec)
            a_v[pl.ds(g * L, L)] = vec
            plsc.store_scatter(e_v, [tr_idx0 + g * epg], jnp.exp(vec))
            return 0

        lax.fori_loop(0, BA * H // L, group_body, 0)
        pltpu.sync_copy(a_v, a_hbm.at[pl.ds(off * H, BA * H)])
        for k in range(H):
            pltpu.sync_copy(e_v.at[pl.ds(k * BA, BA)],
                            e_hbm.at[pl.ds(k * E + off, BA)])
        return 0

    lax.fori_loop(0, EW // BA, blk_body, 0)


def _agg_body(E, N, Np, BB, ZR, H, HPC, U, UW,
              wn_hbm, ii_hbm, io_hbm, e_hbm, num_hbm,
              accum, zero_v, ii_v, io_v, gi_v, e_v, w_v, sem):
    c = lax.axis_index("c")
    s = lax.axis_index("s")
    ES = E // NS
    RZ = Np // NS

    def zv_body(i, _):
        zero_v[i // (UW // L), pl.ds((i % (UW // L)) * L, L)] = (
            jnp.zeros((L,), jnp.float32))
        return 0

    lax.fori_loop(0, ZR * (UW // L), zv_body, 0)

    for kl in range(HPC):
        head = c * HPC + kl

        def zc_body(i, _):
            pltpu.sync_copy(zero_v, accum.at[pl.ds(s * RZ + i * ZR, ZR)])
            return 0

        lax.fori_loop(0, RZ // ZR, zc_body, 0)
        plsc.subcore_barrier()

        def blk_body(b, _):
            off = s * ES + b * BB
            pltpu.sync_copy(ii_hbm.at[pl.ds(off, BB)], ii_v)
            pltpu.sync_copy(io_hbm.at[pl.ds(off, BB)], io_v)
            pltpu.sync_copy(e_hbm.at[pl.ds(head * E + off, BB)], e_v)

            def gi_body(i, _):
                gi_v[pl.ds(i * L, L)] = io_v[pl.ds(i * L, L)] + head * N
                return 0

            lax.fori_loop(0, BB // L, gi_body, 0)
            pltpu.async_copy(wn_hbm.at[gi_v], w_v, sem).wait()

            def scale_body(gg, _):
                g16 = e_v[pl.ds(gg * L, L)]
                for eo in range(L):
                    e = gg * L + eo
                    for j in range(UW // L):
                        sl = pl.ds(j * L, L)
                        w_v[e, sl] = w_v[e, sl] * g16[eo]
                return 0

            lax.fori_loop(0, BB // L, scale_body, 0)
            pltpu.sync_copy(w_v, accum.at[ii_v], add=True)
            return 0

        lax.fori_loop(0, ES // BB, blk_body, 0)
        plsc.subcore_barrier()
        pltpu.sync_copy(accum.at[pl.ds(s * RZ, RZ)],
                        num_hbm.at[pl.ds(head * Np + s * RZ, RZ)])


def _post_body(U, H, num_ref, out_ref):
    for k in range(H):
        v = num_ref[k, :, :U]
        d = num_ref[k, :, U:U + 1]
        out_ref[:, k * U:(k + 1) * U] = _leaky(v / (d + 1e-16))


def kernel(node, edge, edge_index, W_lin, b_lin, W_att, b_att, W_alpha):
    N, D = node.shape
    E = edge_index.shape[1]
    H, _, U = W_lin.shape
    HU = H * U
    UW = U + L
    HPC = H // NC

    w_cat = jnp.concatenate([
        W_lin.transpose(1, 0, 2).reshape(D, HU),
        W_att[:, :D, :].transpose(1, 0, 2).reshape(D, HU),
        W_att[:, D:, :].transpose(1, 0, 2).reshape(D, HU),
    ], axis=1)
    bl = b_lin.reshape(1, HU)
    ba = b_att.reshape(1, HU)
    alpha = W_alpha.reshape(H, U)
    idx_in = edge_index[0]
    idx_out = edge_index[1]

    RB = 1000
    grid = (N // RB,)
    wn, pin, pout = pl.pallas_call(
        functools.partial(_pre_body, U, UW, H),
        grid=grid,
        in_specs=[
            pl.BlockSpec((RB, D), lambda i: (i, 0)),
            pl.BlockSpec((D, 3 * HU), lambda i: (0, 0)),
            pl.BlockSpec((1, HU), lambda i: (0, 0)),
            pl.BlockSpec((1, HU), lambda i: (0, 0)),
        ],
        out_specs=[
            pl.BlockSpec((H, RB, UW), lambda i: (0, i, 0)),
            pl.BlockSpec((RB, HU), lambda i: (i, 0)),
            pl.BlockSpec((RB, HU), lambda i: (i, 0)),
        ],
        out_shape=[
            jax.ShapeDtypeStruct((H, N, UW), jnp.float32),
            jax.ShapeDtypeStruct((N, HU), jnp.float32),
            jax.ShapeDtypeStruct((N, HU), jnp.float32),
        ],
    )(node, w_cat, bl, ba)

    EW = E // NW
    BA = 80
    mesh = plsc.VectorSubcoreMesh(core_axis_name="c", subcore_axis_name="s",
                                  num_cores=NC, num_subcores=NS)
    a_flat, exp_flat = pl.kernel(
        functools.partial(_score_body, E, EW, BA, H, U, UW),
        out_type=[
            jax.ShapeDtypeStruct((E * H,), jnp.float32),
            jax.ShapeDtypeStruct((H * E,), jnp.float32),
        ],
        mesh=mesh,
        scratch_types=[
            pltpu.VMEM((BA,), jnp.int32),
            pltpu.VMEM((BA,), jnp.int32),
            pltpu.VMEM((BA, HU), jnp.float32),
            pltpu.VMEM((BA, HU), jnp.float32),
            pltpu.VMEM((BA * H,), jnp.float32),
            pltpu.VMEM((H * BA,), jnp.float32),
            pltpu.VMEM((H, U), jnp.float32),
            pltpu.SemaphoreType.DMA,
            pltpu.SemaphoreType.DMA,
        ],
        compiler_params=pltpu.CompilerParams(needs_layout_passes=False),
    )(pin, pout, idx_in, idx_out, alpha)

    BB = 80
    ZR = 64
    Np = -(-N // (64 * NS)) * (64 * NS)
    wn_flat = wn.reshape(H * N, UW)
    num = pl.kernel(
        functools.partial(_agg_body, E, N, Np, BB, ZR, H, HPC, U, UW),
        out_type=jax.ShapeDtypeStruct((H * Np, UW), jnp.float32),
        mesh=mesh,
        scratch_types=[
            pltpu.VMEM_SHARED((Np, UW), jnp.float32),
            pltpu.VMEM((ZR, UW), jnp.float32),
            pltpu.VMEM((BB,), jnp.int32),
            pltpu.VMEM((BB,), jnp.int32),
            pltpu.VMEM((BB,), jnp.int32),
            pltpu.VMEM((BB,), jnp.float32),
            pltpu.VMEM((BB, UW), jnp.float32),
            pltpu.SemaphoreType.DMA,
        ],
        compiler_params=pltpu.CompilerParams(
            use_tc_tiling_on_sc=False, needs_layout_passes=False),
    )(wn_flat, idx_in, idx_out, exp_flat)

    h_is = pl.pallas_call(
        functools.partial(_post_body, U, H),
        grid=grid,
        in_specs=[pl.BlockSpec((H, RB, UW), lambda i: (0, i, 0))],
        out_specs=pl.BlockSpec((RB, HU), lambda i: (i, 0)),
        out_shape=jax.ShapeDtypeStruct((N, HU), jnp.float32),
    )(num.reshape(H, Np, UW))

    a_ijs = a_flat.reshape(E, H, 1)
    return h_is, a_ijs


# speedup vs baseline: 6.8660x; 1.3309x over previous
import functools

import jax
import jax.numpy as jnp
from jax import lax
from jax.experimental import pallas as pl
from jax.experimental.pallas import tpu as pltpu
from jax.experimental.pallas import tpu_sc as plsc

NC = 2
NS = 16
L = 16
NW = NC * NS


def _leaky(x):
    return jnp.where(x > 0, x, 0.2 * x)


def _pre_body(U, UW, H, node_ref, wcat_ref, bl_ref, ba_ref,
              wn_ref, pin_ref, pout_ref):
    x = node_ref[...]
    HU = H * U
    p = jnp.dot(x, wcat_ref[...], preferred_element_type=jnp.float32)
    wn = _leaky(p[:, :HU] + bl_ref[...])
    pin_ref[...] = p[:, HU:2 * HU] + ba_ref[...]
    pout_ref[...] = p[:, 2 * HU:]
    rb = x.shape[0]
    pad = (lax.broadcasted_iota(jnp.int32, (rb, UW - U), 1) == 0)
    for k in range(H):
        wn_ref[k, :, :U] = wn[:, k * U:(k + 1) * U]
        wn_ref[k, :, U:] = pad.astype(jnp.float32)


def _score_body(E, EW, BA, H, U, UW,
                pin_hbm, pout_hbm, ii_hbm, io_hbm, al_hbm, a_hbm, e_hbm,
                ii_v, io_v, pin_v, pout_v, a_v, e_v, al_v, sem1, sem2):
    wid = lax.axis_index("s") * NC + lax.axis_index("c")
    base = wid * EW
    pltpu.sync_copy(al_hbm, al_v)

    def blk_body(b, _):
        off = base + b * BA
        pltpu.sync_copy(ii_hbm.at[pl.ds(off, BA)], ii_v)
        pltpu.sync_copy(io_hbm.at[pl.ds(off, BA)], io_v)
        cp1 = pltpu.async_copy(pin_hbm.at[ii_v], pin_v, sem1)
        cp2 = pltpu.async_copy(pout_hbm.at[io_v], pout_v, sem2)
        cp1.wait()
        cp2.wait()

        epg = L // H
        lanes = lax.iota(jnp.int32, L)
        tr_idx0 = (lanes % H) * BA + lanes // H

        def group_body(g, _):
            vec = jnp.zeros((L,), jnp.float32)
            for eo in range(epg):
                e = g * epg + eo
                for k in range(H):
                    acc = jnp.zeros((L,), jnp.float32)
                    for c in range(U // L):
                        sl = pl.ds(k * U + c * L, L)
                        x = pin_v[e, sl] + pout_v[e, sl]
                        acc = acc + _leaky(x) * al_v[k, pl.ds(c * L, L)]
                    vec = jnp.where(lanes == eo * H + k, jnp.sum(acc), vec)
            a_v[pl.ds(g * L, L)] = vec
            plsc.store_scatter(e_v, [tr_idx0 + g * epg], jnp.exp(vec))
            return 0

        lax.fori_loop(0, BA * H // L, group_body, 0)
        pltpu.sync_copy(a_v, a_hbm.at[pl.ds(off * H, BA * H)])
        for k in range(H):
            pltpu.sync_copy(e_v.at[pl.ds(k * BA, BA)],
                            e_hbm.at[pl.ds(k * E + off, BA)])
        return 0

    lax.fori_loop(0, EW // BA, blk_body, 0)


def _agg_body(E, N, Np, BB, NBC, H, HPC, U, UW,
              wn_hbm, ii_hbm, io_hbm, e_hbm, num_hbm,
              accum, ii_v, io_v, e_v, ib0, ib1, gi0, gi1, w0, w1,
              sg0, sg1, ss0, ss1):
    c = lax.axis_index("c")
    s = lax.axis_index("s")
    ES = E // NS
    RZ = Np // NS
    CH = NBC * BB
    SCB = BB * UW * 4
    GRP = BB // L
    CW = UW // L
    bufs = ((ib0, gi0, w0, sg0, ss0), (ib1, gi1, w1, sg1, ss1))

    for kl in range(HPC):
        head = c * HPC + kl

        def zw_body(i, _):
            w0[i // CW, pl.ds((i % CW) * L, L)] = jnp.zeros((L,), jnp.float32)
            return 0

        lax.fori_loop(0, BB * CW, zw_body, 0)

        def zc_body(i, _):
            pltpu.sync_copy(w0, accum.at[pl.ds(s * RZ + i * BB, BB)])
            return 0

        lax.fori_loop(0, RZ // BB, zc_body, 0)
        plsc.subcore_barrier()

        def chunk_body(chk, _):
            coff = s * ES + chk * CH
            pltpu.sync_copy(ii_hbm.at[pl.ds(coff, CH)], ii_v)
            pltpu.sync_copy(io_hbm.at[pl.ds(coff, CH)], io_v)
            pltpu.sync_copy(e_hbm.at[pl.ds(head * E + coff, CH)], e_v)

            def pair_body(bp, _):
                cps = []
                for p, (ib, gi, w, sg, ss) in enumerate(bufs):
                    bo = (bp * 2 + p) * BB
                    for g in range(GRP):
                        sl = pl.ds(g * L, L)
                        ib[sl] = ii_v[pl.ds(bo + g * L, L)]
                        gi[sl] = io_v[pl.ds(bo + g * L, L)] + head * N
                    cps.append(pltpu.async_copy(wn_hbm.at[gi], w, sg))
                for p, (ib, gi, w, sg, ss) in enumerate(bufs):
                    bo = (bp * 2 + p) * BB
                    cps[p].wait()

                    def scale_body(gg, _):
                        g16 = e_v[pl.ds(bo + gg * L, L)]
                        for eo in range(L):
                            for j in range(CW):
                                sl = pl.ds(j * L, L)
                                w[gg * L + eo, sl] = w[gg * L + eo, sl] * g16[eo]
                        return 0

                    lax.fori_loop(0, GRP, scale_body, 0)
                    pltpu.sync_copy(w, accum.at[ib], add=True)
                return 0

            lax.fori_loop(0, NBC // 2, pair_body, 0)
            return 0

        lax.fori_loop(0, ES // CH, chunk_body, 0)
        plsc.subcore_barrier()
        pltpu.sync_copy(accum.at[pl.ds(s * RZ, RZ)],
                        num_hbm.at[pl.ds(head * Np + s * RZ, RZ)])


def _post_body(U, H, num_ref, out_ref):
    for k in range(H):
        v = num_ref[k, :, :U]
        d = num_ref[k, :, U:U + 1]
        out_ref[:, k * U:(k + 1) * U] = _leaky(v / (d + 1e-16))


def kernel(node, edge, edge_index, W_lin, b_lin, W_att, b_att, W_alpha):
    N, D = node.shape
    E = edge_index.shape[1]
    H, _, U = W_lin.shape
    HU = H * U
    UW = U + L
    HPC = H // NC

    w_cat = jnp.concatenate([
        W_lin.transpose(1, 0, 2).reshape(D, HU),
        W_att[:, :D, :].transpose(1, 0, 2).reshape(D, HU),
        W_att[:, D:, :].transpose(1, 0, 2).reshape(D, HU),
    ], axis=1)
    bl = b_lin.reshape(1, HU)
    ba = b_att.reshape(1, HU)
    alpha = W_alpha.reshape(H, U)
    idx_in = edge_index[0]
    idx_out = edge_index[1]

    RB = 1000
    grid = (N // RB,)
    wn, pin, pout = pl.pallas_call(
        functools.partial(_pre_body, U, UW, H),
        grid=grid,
        in_specs=[
            pl.BlockSpec((RB, D), lambda i: (i, 0)),
            pl.BlockSpec((D, 3 * HU), lambda i: (0, 0)),
            pl.BlockSpec((1, HU), lambda i: (0, 0)),
            pl.BlockSpec((1, HU), lambda i: (0, 0)),
        ],
        out_specs=[
            pl.BlockSpec((H, RB, UW), lambda i: (0, i, 0)),
            pl.BlockSpec((RB, HU), lambda i: (i, 0)),
            pl.BlockSpec((RB, HU), lambda i: (i, 0)),
        ],
        out_shape=[
            jax.ShapeDtypeStruct((H, N, UW), jnp.float32),
            jax.ShapeDtypeStruct((N, HU), jnp.float32),
            jax.ShapeDtypeStruct((N, HU), jnp.float32),
        ],
    )(node, w_cat, bl, ba)

    EW = E // NW
    BA = 80
    mesh = plsc.VectorSubcoreMesh(core_axis_name="c", subcore_axis_name="s",
                                  num_cores=NC, num_subcores=NS)
    a_flat, exp_flat = pl.kernel(
        functools.partial(_score_body, E, EW, BA, H, U, UW),
        out_type=[
            jax.ShapeDtypeStruct((E * H,), jnp.float32),
            jax.ShapeDtypeStruct((H * E,), jnp.float32),
        ],
        mesh=mesh,
        scratch_types=[
            pltpu.VMEM((BA,), jnp.int32),
            pltpu.VMEM((BA,), jnp.int32),
            pltpu.VMEM((BA, HU), jnp.float32),
            pltpu.VMEM((BA, HU), jnp.float32),
            pltpu.VMEM((BA * H,), jnp.float32),
            pltpu.VMEM((H * BA,), jnp.float32),
            pltpu.VMEM((H, U), jnp.float32),
            pltpu.SemaphoreType.DMA,
            pltpu.SemaphoreType.DMA,
        ],
        compiler_params=pltpu.CompilerParams(needs_layout_passes=False),
    )(pin, pout, idx_in, idx_out, alpha)

    BB = 80
    NBC = 10
    Np = -(-N // (64 * NS)) * (64 * NS)
    wn_flat = wn.reshape(H * N, UW)
    num = pl.kernel(
        functools.partial(_agg_body, E, N, Np, BB, NBC, H, HPC, U, UW),
        out_type=jax.ShapeDtypeStruct((H * Np, UW), jnp.float32),
        mesh=mesh,
        scratch_types=[
            pltpu.VMEM_SHARED((Np, UW), jnp.float32),
            pltpu.VMEM((NBC * BB,), jnp.int32),
            pltpu.VMEM((NBC * BB,), jnp.int32),
            pltpu.VMEM((NBC * BB,), jnp.float32),
            pltpu.VMEM((BB,), jnp.int32),
            pltpu.VMEM((BB,), jnp.int32),
            pltpu.VMEM((BB,), jnp.int32),
            pltpu.VMEM((BB,), jnp.int32),
            pltpu.VMEM((BB, UW), jnp.float32),
            pltpu.VMEM((BB, UW), jnp.float32),
            pltpu.SemaphoreType.DMA,
            pltpu.SemaphoreType.DMA,
            pltpu.SemaphoreType.DMA,
            pltpu.SemaphoreType.DMA,
        ],
        compiler_params=pltpu.CompilerParams(
            use_tc_tiling_on_sc=False, needs_layout_passes=False),
    )(wn_flat, idx_in, idx_out, exp_flat)

    h_is = pl.pallas_call(
        functools.partial(_post_body, U, H),
        grid=grid,
        in_specs=[pl.BlockSpec((H, RB, UW), lambda i: (0, i, 0))],
        out_specs=pl.BlockSpec((RB, HU), lambda i: (i, 0)),
        out_shape=jax.ShapeDtypeStruct((N, HU), jnp.float32),
    )(num.reshape(H, Np, UW))

    a_ijs = a_flat.reshape(E, H, 1)
    return h_is, a_ijs


# speedup vs baseline: 9.2482x; 1.3470x over previous
import functools

import jax
import jax.numpy as jnp
from jax import lax
from jax.experimental import pallas as pl
from jax.experimental.pallas import tpu as pltpu
from jax.experimental.pallas import tpu_sc as plsc

NC = 2
NS = 16
L = 16
NW = NC * NS


def _leaky(x):
    return jnp.where(x > 0, x, 0.2 * x)


def _pre_body(U, UW, H, node_ref, wcat_ref, bl_ref, ba_ref,
              wn_ref, pin_ref, pout_ref):
    x = node_ref[...]
    HU = H * U
    p = jnp.dot(x, wcat_ref[...], preferred_element_type=jnp.float32)
    wn = _leaky(p[:, :HU] + bl_ref[...])
    pin_ref[...] = p[:, HU:2 * HU] + ba_ref[...]
    pout_ref[...] = p[:, 2 * HU:]
    rb = x.shape[0]
    pad = (lax.broadcasted_iota(jnp.int32, (rb, UW - U), 1) == 0)
    for k in range(H):
        wn_ref[k, :, :U] = wn[:, k * U:(k + 1) * U]
        wn_ref[k, :, U:] = pad.astype(jnp.float32)


def _score_body(E, EW, BA, OB, H, U, UW,
                pin_hbm, pout_hbm, ii_hbm, io_hbm, al_hbm, a_hbm, e_hbm,
                ii_v, io_v, al_v, pin0, pout0, pin1, pout1, a_v, e_v,
                sg0, sg1):
    wid = lax.axis_index("s") * NC + lax.axis_index("c")
    base = wid * EW
    HU = H * U
    pltpu.sync_copy(al_hbm, al_v)
    pltpu.sync_copy(ii_hbm.at[pl.ds(base, EW)], ii_v)
    pltpu.sync_copy(io_hbm.at[pl.ds(base, EW)], io_v)
    NBLK = EW // BA
    NPAIR = NBLK // 2
    epg = L // H
    lanes = lax.iota(jnp.int32, L)
    tr_idx0 = (lanes % H) * (OB * BA) + lanes // H
    bufs = ((pin0, pout0, sg0), (pin1, pout1, sg1))
    GB = BA * HU * 4

    def start_gather(b, p):
        pin_b, pout_b, sg = bufs[p]
        pltpu.async_copy(pin_hbm.at[ii_v.at[pl.ds(b * BA, BA)]], pin_b, sg)
        pltpu.async_copy(pout_hbm.at[io_v.at[pl.ds(b * BA, BA)]], pout_b, sg)

    def wait_gather(p):
        pin_b, pout_b, sg = bufs[p]
        pltpu.make_async_copy(pin_hbm.at[pl.ds(0, BA)], pin_b, sg).wait()
        pltpu.make_async_copy(pin_hbm.at[pl.ds(0, BA)], pout_b, sg).wait()

    def compute_block(b, p):
        pin_b, pout_b, _ = bufs[p]
        q = b % OB

        def group_body(g, _):
            vec = jnp.zeros((L,), jnp.float32)
            for eo in range(epg):
                e = g * epg + eo
                for k in range(H):
                    acc = jnp.zeros((L,), jnp.float32)
                    for c in range(U // L):
                        sl = pl.ds(k * U + c * L, L)
                        x = pin_b[e, sl] + pout_b[e, sl]
                        acc = acc + _leaky(x) * al_v[k, pl.ds(c * L, L)]
                    vec = jnp.where(lanes == eo * H + k, jnp.sum(acc), vec)
            a_v[pl.ds(q * BA * H + g * L, L)] = vec
            plsc.store_scatter(e_v, [tr_idx0 + (q * BA + g * epg)],
                               jnp.exp(vec))
            return 0

        lax.fori_loop(0, BA * H // L, group_body, 0)

    start_gather(0, 0)
    start_gather(1, 1)

    def pair_body(bp, _):
        b0 = bp * 2
        for p in range(2):
            b = b0 + p
            wait_gather(p)
            compute_block(b, p)

            @pl.when(bp < NPAIR - 1)
            def _():
                start_gather(b + 2, p)

        @pl.when(bp % (OB // 2) == OB // 2 - 1)
        def _():
            grp0 = base + (b0 + 2 - OB) * BA
            pltpu.sync_copy(a_v, a_hbm.at[pl.ds(grp0 * H, OB * BA * H)])
            for k in range(H):
                pltpu.sync_copy(e_v.at[pl.ds(k * OB * BA, OB * BA)],
                                e_hbm.at[pl.ds(k * E + grp0, OB * BA)])
        return 0

    lax.fori_loop(0, NPAIR, pair_body, 0)


def _agg_body(E, N, Np, BB, NBC, H, HPC, U, UW,
              wn_hbm, ii_hbm, io_hbm, e_hbm, num_hbm,
              accum, ii_v, io_v, e_v, ib0, ib1, gi0, gi1, w0, w1,
              sg0, sg1, ss0, ss1):
    c = lax.axis_index("c")
    s = lax.axis_index("s")
    ES = E // NS
    RZ = Np // NS
    CH = NBC * BB
    SCB = BB * UW * 4
    GRP = BB // L
    CW = UW // L
    bufs = ((ib0, gi0, w0, sg0, ss0), (ib1, gi1, w1, sg1, ss1))

    for kl in range(HPC):
        head = c * HPC + kl

        def zw_body(i, _):
            w0[i // CW, pl.ds((i % CW) * L, L)] = jnp.zeros((L,), jnp.float32)
            return 0

        lax.fori_loop(0, BB * CW, zw_body, 0)

        def zc_body(i, _):
            pltpu.sync_copy(w0, accum.at[pl.ds(s * RZ + i * BB, BB)])
            return 0

        lax.fori_loop(0, RZ // BB, zc_body, 0)
        plsc.subcore_barrier()

        def chunk_body(chk, _):
            coff = s * ES + chk * CH
            pltpu.sync_copy(ii_hbm.at[pl.ds(coff, CH)], ii_v)
            pltpu.sync_copy(io_hbm.at[pl.ds(coff, CH)], io_v)
            pltpu.sync_copy(e_hbm.at[pl.ds(head * E + coff, CH)], e_v)

            def pair_body(bp, _):
                cps = []
                for p, (ib, gi, w, sg, ss) in enumerate(bufs):
                    bo = (bp * 2 + p) * BB
                    for g in range(GRP):
                        sl = pl.ds(g * L, L)
                        ib[sl] = ii_v[pl.ds(bo + g * L, L)]
                        gi[sl] = io_v[pl.ds(bo + g * L, L)] + head * N
                    cps.append(pltpu.async_copy(wn_hbm.at[gi], w, sg))
                for p, (ib, gi, w, sg, ss) in enumerate(bufs):
                    bo = (bp * 2 + p) * BB
                    cps[p].wait()

                    def scale_body(gg, _):
                        g16 = e_v[pl.ds(bo + gg * L, L)]
                        for eo in range(L):
                            for j in range(CW):
                                sl = pl.ds(j * L, L)
                                w[gg * L + eo, sl] = w[gg * L + eo, sl] * g16[eo]
                        return 0

                    lax.fori_loop(0, GRP, scale_body, 0)
                    pltpu.sync_copy(w, accum.at[ib], add=True)
                return 0

            lax.fori_loop(0, NBC // 2, pair_body, 0)
            return 0

        lax.fori_loop(0, ES // CH, chunk_body, 0)
        plsc.subcore_barrier()
        pltpu.sync_copy(accum.at[pl.ds(s * RZ, RZ)],
                        num_hbm.at[pl.ds(head * Np + s * RZ, RZ)])


def _post_body(U, H, num_ref, out_ref):
    for k in range(H):
        v = num_ref[k, :, :U]
        d = num_ref[k, :, U:U + 1]
        out_ref[:, k * U:(k + 1) * U] = _leaky(v / (d + 1e-16))


def kernel(node, edge, edge_index, W_lin, b_lin, W_att, b_att, W_alpha):
    N, D = node.shape
    E = edge_index.shape[1]
    H, _, U = W_lin.shape
    HU = H * U
    UW = U + L
    HPC = H // NC

    w_cat = jnp.concatenate([
        W_lin.transpose(1, 0, 2).reshape(D, HU),
        W_att[:, :D, :].transpose(1, 0, 2).reshape(D, HU),
        W_att[:, D:, :].transpose(1, 0, 2).reshape(D, HU),
    ], axis=1)
    bl = b_lin.reshape(1, HU)
    ba = b_att.reshape(1, HU)
    alpha = W_alpha.reshape(H, U)
    idx_in = edge_index[0]
    idx_out = edge_index[1]

    RB = 1000
    grid = (N // RB,)
    wn, pin, pout = pl.pallas_call(
        functools.partial(_pre_body, U, UW, H),
        grid=grid,
        in_specs=[
            pl.BlockSpec((RB, D), lambda i: (i, 0)),
            pl.BlockSpec((D, 3 * HU), lambda i: (0, 0)),
            pl.BlockSpec((1, HU), lambda i: (0, 0)),
            pl.BlockSpec((1, HU), lambda i: (0, 0)),
        ],
        out_specs=[
            pl.BlockSpec((H, RB, UW), lambda i: (0, i, 0)),
            pl.BlockSpec((RB, HU), lambda i: (i, 0)),
            pl.BlockSpec((RB, HU), lambda i: (i, 0)),
        ],
        out_shape=[
            jax.ShapeDtypeStruct((H, N, UW), jnp.float32),
            jax.ShapeDtypeStruct((N, HU), jnp.float32),
            jax.ShapeDtypeStruct((N, HU), jnp.float32),
        ],
    )(node, w_cat, bl, ba)

    EW = E // NW
    BA = 40
    OB = 10
    mesh = plsc.VectorSubcoreMesh(core_axis_name="c", subcore_axis_name="s",
                                  num_cores=NC, num_subcores=NS)
    a_flat, exp_flat = pl.kernel(
        functools.partial(_score_body, E, EW, BA, OB, H, U, UW),
        out_type=[
            jax.ShapeDtypeStruct((E * H,), jnp.float32),
            jax.ShapeDtypeStruct((H * E,), jnp.float32),
        ],
        mesh=mesh,
        scratch_types=[
            pltpu.VMEM((EW,), jnp.int32),
            pltpu.VMEM((EW,), jnp.int32),
            pltpu.VMEM((H, U), jnp.float32),
            pltpu.VMEM((BA, HU), jnp.float32),
            pltpu.VMEM((BA, HU), jnp.float32),
            pltpu.VMEM((BA, HU), jnp.float32),
            pltpu.VMEM((BA, HU), jnp.float32),
            pltpu.VMEM((OB * BA * H,), jnp.float32),
            pltpu.VMEM((H * OB * BA,), jnp.float32),
            pltpu.SemaphoreType.DMA,
            pltpu.SemaphoreType.DMA,
        ],
        compiler_params=pltpu.CompilerParams(needs_layout_passes=False),
    )(pin, pout, idx_in, idx_out, alpha)

    BB = 80
    NBC = 10
    Np = -(-N // (64 * NS)) * (64 * NS)
    wn_flat = wn.reshape(H * N, UW)
    num = pl.kernel(
        functools.partial(_agg_body, E, N, Np, BB, NBC, H, HPC, U, UW),
        out_type=jax.ShapeDtypeStruct((H * Np, UW), jnp.float32),
        mesh=mesh,
        scratch_types=[
            pltpu.VMEM_SHARED((Np, UW), jnp.float32),
            pltpu.VMEM((NBC * BB,), jnp.int32),
            pltpu.VMEM((NBC * BB,), jnp.int32),
            pltpu.VMEM((NBC * BB,), jnp.float32),
            pltpu.VMEM((BB,), jnp.int32),
            pltpu.VMEM((BB,), jnp.int32),
            pltpu.VMEM((BB,), jnp.int32),
            pltpu.VMEM((BB,), jnp.int32),
            pltpu.VMEM((BB, UW), jnp.float32),
            pltpu.VMEM((BB, UW), jnp.float32),
            pltpu.SemaphoreType.DMA,
            pltpu.SemaphoreType.DMA,
            pltpu.SemaphoreType.DMA,
            pltpu.SemaphoreType.DMA,
        ],
        compiler_params=pltpu.CompilerParams(
            use_tc_tiling_on_sc=False, needs_layout_passes=False),
    )(wn_flat, idx_in, idx_out, exp_flat)

    h_is = pl.pallas_call(
        functools.partial(_post_body, U, H),
        grid=grid,
        in_specs=[pl.BlockSpec((H, RB, UW), lambda i: (0, i, 0))],
        out_specs=pl.BlockSpec((RB, HU), lambda i: (i, 0)),
        out_shape=jax.ShapeDtypeStruct((N, HU), jnp.float32),
    )(num.reshape(H, Np, UW))

    a_ijs = a_flat.reshape(E, H, 1)
    return h_is, a_ijs


# speedup vs baseline: 10.5348x; 1.1391x over previous
import functools

import jax
import jax.numpy as jnp
from jax import lax
from jax.experimental import pallas as pl
from jax.experimental.pallas import tpu as pltpu
from jax.experimental.pallas import tpu_sc as plsc

NC = 2
NS = 16
L = 16
NW = NC * NS


def _leaky(x):
    return jnp.where(x > 0, x, 0.2 * x)


def _pre_body(U, UW, H, node_ref, wcat_ref, bl_ref, ba_ref,
              wn_ref, pin_ref, pout_ref):
    x = node_ref[...]
    HU = H * U
    p = jnp.dot(x, wcat_ref[...], preferred_element_type=jnp.float32)
    wn = _leaky(p[:, :HU] + bl_ref[...])
    pin_ref[...] = p[:, HU:2 * HU] + ba_ref[...]
    pout_ref[...] = p[:, 2 * HU:]
    rb = x.shape[0]
    pad = (lax.broadcasted_iota(jnp.int32, (rb, UW - U), 1) == 0)
    for k in range(H):
        wn_ref[k, :, :U] = wn[:, k * U:(k + 1) * U]
        wn_ref[k, :, U:] = pad.astype(jnp.float32)


def _score_body(E, EW, BA, OB, H, U, UW,
                pin_hbm, pout_hbm, ii_hbm, io_hbm, al_hbm, a_hbm, e_hbm,
                ii_v, io_v, al_v, pin0, pout0, pin1, pout1, a_v, e_v,
                sg0, sg1):
    wid = lax.axis_index("s") * NC + lax.axis_index("c")
    base = wid * EW
    HU = H * U
    pltpu.sync_copy(al_hbm, al_v)
    pltpu.sync_copy(ii_hbm.at[pl.ds(base, EW)], ii_v)
    pltpu.sync_copy(io_hbm.at[pl.ds(base, EW)], io_v)
    NBLK = EW // BA
    NPAIR = NBLK // 2
    epg = L // H
    lanes = lax.iota(jnp.int32, L)
    tr_idx0 = (lanes % H) * (OB * BA) + lanes // H
    bufs = ((pin0, pout0, sg0), (pin1, pout1, sg1))
    GB = BA * HU * 4

    def start_gather(b, p):
        pin_b, pout_b, sg = bufs[p]
        pltpu.async_copy(pin_hbm.at[ii_v.at[pl.ds(b * BA, BA)]], pin_b, sg)
        pltpu.async_copy(pout_hbm.at[io_v.at[pl.ds(b * BA, BA)]], pout_b, sg)

    def wait_gather(p):
        pin_b, pout_b, sg = bufs[p]
        pltpu.make_async_copy(pin_hbm.at[pl.ds(0, BA)], pin_b, sg).wait()
        pltpu.make_async_copy(pin_hbm.at[pl.ds(0, BA)], pout_b, sg).wait()

    def compute_block(b, p):
        pin_b, pout_b, _ = bufs[p]
        q = b % OB

        def group_body(g, _):
            vec = jnp.zeros((L,), jnp.float32)
            for eo in range(epg):
                e = g * epg + eo
                for k in range(H):
                    acc = jnp.zeros((L,), jnp.float32)
                    for c in range(U // L):
                        sl = pl.ds(k * U + c * L, L)
                        x = pin_b[e, sl] + pout_b[e, sl]
                        acc = acc + _leaky(x) * al_v[k, pl.ds(c * L, L)]
                    vec = jnp.where(lanes == eo * H + k, jnp.sum(acc), vec)
            a_v[pl.ds(q * BA * H + g * L, L)] = vec
            plsc.store_scatter(e_v, [tr_idx0 + (q * BA + g * epg)],
                               jnp.exp(vec))
            return 0

        lax.fori_loop(0, BA * H // L, group_body, 0)

    start_gather(0, 0)
    start_gather(1, 1)

    def pair_body(bp, _):
        b0 = bp * 2
        for p in range(2):
            b = b0 + p
            wait_gather(p)
            compute_block(b, p)

            @pl.when(bp < NPAIR - 1)
            def _():
                start_gather(b + 2, p)

        @pl.when(bp % (OB // 2) == OB // 2 - 1)
        def _():
            grp0 = base + (b0 + 2 - OB) * BA
            pltpu.sync_copy(a_v, a_hbm.at[pl.ds(grp0 * H, OB * BA * H)])
            for k in range(H):
                pltpu.sync_copy(e_v.at[pl.ds(k * OB * BA, OB * BA)],
                                e_hbm.at[pl.ds(k * E + grp0, OB * BA)])
        return 0

    lax.fori_loop(0, NPAIR, pair_body, 0)


def _agg_body(E, N, Np, BB, NBC, H, HPC, U, UW,
              wn_hbm, ii_hbm, io_hbm, e_hbm, num_hbm,
              accum, ii_v, io_v, e_v, ib0, ib1, gi0, gi1, w0, w1,
              sg0, sg1, ss0, ss1):
    c = lax.axis_index("c")
    s = lax.axis_index("s")
    ES = E // NS
    RZ = Np // NS
    CH = NBC * BB
    SCB = BB * UW * 4
    GRP = BB // L
    CW = UW // L
    bufs = ((ib0, gi0, w0, sg0, ss0), (ib1, gi1, w1, sg1, ss1))

    for kl in range(HPC):
        head = c * HPC + kl

        def zw_body(i, _):
            w0[i // CW, pl.ds((i % CW) * L, L)] = jnp.zeros((L,), jnp.float32)
            return 0

        lax.fori_loop(0, BB * CW, zw_body, 0)

        def zc_body(i, _):
            pltpu.sync_copy(w0, accum.at[pl.ds(s * RZ + i * BB, BB)])
            return 0

        lax.fori_loop(0, RZ // BB, zc_body, 0)
        plsc.subcore_barrier()

        def build_and_gather(p, bo):
            ib, gi, w, sg, ss = bufs[p]
            for g in range(GRP):
                sl = pl.ds(g * L, L)
                ib[sl] = ii_v[pl.ds(bo + g * L, L)]
                gi[sl] = io_v[pl.ds(bo + g * L, L)] + head * N
            pltpu.async_copy(wn_hbm.at[gi], w, sg)

        def wait_gather(p):
            ib, gi, w, sg, ss = bufs[p]
            pltpu.make_async_copy(wn_hbm.at[pl.ds(0, BB)], w, sg).wait()

        def scale(p, bo):
            ib, gi, w, sg, ss = bufs[p]

            def scale_body(gg, _):
                g16 = e_v[pl.ds(bo + gg * L, L)]
                for eo in range(L):
                    for j in range(CW):
                        sl = pl.ds(j * L, L)
                        w[gg * L + eo, sl] = w[gg * L + eo, sl] * g16[eo]
                return 0

            lax.fori_loop(0, GRP, scale_body, 0)

        def scat(p):
            ib, gi, w, sg, ss = bufs[p]
            return pltpu.async_copy(w, accum.at[ib], ss, add=True)

        def chunk_body(chk, _):
            coff = s * ES + chk * CH
            pltpu.sync_copy(ii_hbm.at[pl.ds(coff, CH)], ii_v)
            pltpu.sync_copy(io_hbm.at[pl.ds(coff, CH)], io_v)
            pltpu.sync_copy(e_hbm.at[pl.ds(head * E + coff, CH)], e_v)
            build_and_gather(0, 0)
            build_and_gather(1, BB)

            def pair_body(bp, _):
                bo0 = bp * 2 * BB
                wait_gather(0)
                scale(0, bo0)
                sc0 = scat(0)
                wait_gather(1)
                scale(1, bo0 + BB)
                sc0.wait()
                build_and_gather(0, bo0 + 2 * BB)
                sc1 = scat(1)
                sc1.wait()
                build_and_gather(1, bo0 + 3 * BB)
                return 0

            lax.fori_loop(0, NBC // 2 - 1, pair_body, 0)
            bo0 = (NBC - 2) * BB
            wait_gather(0)
            scale(0, bo0)
            sc0 = scat(0)
            wait_gather(1)
            scale(1, bo0 + BB)
            sc0.wait()
            sc1 = scat(1)
            sc1.wait()
            return 0

        lax.fori_loop(0, ES // CH, chunk_body, 0)
        plsc.subcore_barrier()
        pltpu.sync_copy(accum.at[pl.ds(s * RZ, RZ)],
                        num_hbm.at[pl.ds(head * Np + s * RZ, RZ)])


def _post_body(U, H, num_ref, out_ref):
    for k in range(H):
        v = num_ref[k, :, :U]
        d = num_ref[k, :, U:U + 1]
        out_ref[:, k * U:(k + 1) * U] = _leaky(v / (d + 1e-16))


def kernel(node, edge, edge_index, W_lin, b_lin, W_att, b_att, W_alpha):
    N, D = node.shape
    E = edge_index.shape[1]
    H, _, U = W_lin.shape
    HU = H * U
    UW = U + L
    HPC = H // NC

    w_cat = jnp.concatenate([
        W_lin.transpose(1, 0, 2).reshape(D, HU),
        W_att[:, :D, :].transpose(1, 0, 2).reshape(D, HU),
        W_att[:, D:, :].transpose(1, 0, 2).reshape(D, HU),
    ], axis=1)
    bl = b_lin.reshape(1, HU)
    ba = b_att.reshape(1, HU)
    alpha = W_alpha.reshape(H, U)
    idx_in = edge_index[0]
    idx_out = edge_index[1]

    RB = 1000
    grid = (N // RB,)
    wn, pin, pout = pl.pallas_call(
        functools.partial(_pre_body, U, UW, H),
        grid=grid,
        in_specs=[
            pl.BlockSpec((RB, D), lambda i: (i, 0)),
            pl.BlockSpec((D, 3 * HU), lambda i: (0, 0)),
            pl.BlockSpec((1, HU), lambda i: (0, 0)),
            pl.BlockSpec((1, HU), lambda i: (0, 0)),
        ],
        out_specs=[
            pl.BlockSpec((H, RB, UW), lambda i: (0, i, 0)),
            pl.BlockSpec((RB, HU), lambda i: (i, 0)),
            pl.BlockSpec((RB, HU), lambda i: (i, 0)),
        ],
        out_shape=[
            jax.ShapeDtypeStruct((H, N, UW), jnp.float32),
            jax.ShapeDtypeStruct((N, HU), jnp.float32),
            jax.ShapeDtypeStruct((N, HU), jnp.float32),
        ],
    )(node, w_cat, bl, ba)

    EW = E // NW
    BA = 40
    OB = 10
    mesh = plsc.VectorSubcoreMesh(core_axis_name="c", subcore_axis_name="s",
                                  num_cores=NC, num_subcores=NS)
    a_flat, exp_flat = pl.kernel(
        functools.partial(_score_body, E, EW, BA, OB, H, U, UW),
        out_type=[
            jax.ShapeDtypeStruct((E * H,), jnp.float32),
            jax.ShapeDtypeStruct((H * E,), jnp.float32),
        ],
        mesh=mesh,
        scratch_types=[
            pltpu.VMEM((EW,), jnp.int32),
            pltpu.VMEM((EW,), jnp.int32),
            pltpu.VMEM((H, U), jnp.float32),
            pltpu.VMEM((BA, HU), jnp.float32),
            pltpu.VMEM((BA, HU), jnp.float32),
            pltpu.VMEM((BA, HU), jnp.float32),
            pltpu.VMEM((BA, HU), jnp.float32),
            pltpu.VMEM((OB * BA * H,), jnp.float32),
            pltpu.VMEM((H * OB * BA,), jnp.float32),
            pltpu.SemaphoreType.DMA,
            pltpu.SemaphoreType.DMA,
        ],
        compiler_params=pltpu.CompilerParams(needs_layout_passes=False),
    )(pin, pout, idx_in, idx_out, alpha)

    BB = 80
    NBC = 10
    Np = -(-N // (64 * NS)) * (64 * NS)
    wn_flat = wn.reshape(H * N, UW)
    num = pl.kernel(
        functools.partial(_agg_body, E, N, Np, BB, NBC, H, HPC, U, UW),
        out_type=jax.ShapeDtypeStruct((H * Np, UW), jnp.float32),
        mesh=mesh,
        scratch_types=[
            pltpu.VMEM_SHARED((Np, UW), jnp.float32),
            pltpu.VMEM((NBC * BB,), jnp.int32),
            pltpu.VMEM((NBC * BB,), jnp.int32),
            pltpu.VMEM((NBC * BB,), jnp.float32),
            pltpu.VMEM((BB,), jnp.int32),
            pltpu.VMEM((BB,), jnp.int32),
            pltpu.VMEM((BB,), jnp.int32),
            pltpu.VMEM((BB,), jnp.int32),
            pltpu.VMEM((BB, UW), jnp.float32),
            pltpu.VMEM((BB, UW), jnp.float32),
            pltpu.SemaphoreType.DMA,
            pltpu.SemaphoreType.DMA,
            pltpu.SemaphoreType.DMA,
            pltpu.SemaphoreType.DMA,
        ],
        compiler_params=pltpu.CompilerParams(
            use_tc_tiling_on_sc=False, needs_layout_passes=False),
    )(wn_flat, idx_in, idx_out, exp_flat)

    h_is = pl.pallas_call(
        functools.partial(_post_body, U, H),
        grid=grid,
        in_specs=[pl.BlockSpec((H, RB, UW), lambda i: (0, i, 0))],
        out_specs=pl.BlockSpec((RB, HU), lambda i: (i, 0)),
        out_shape=jax.ShapeDtypeStruct((N, HU), jnp.float32),
    )(num.reshape(H, Np, UW))

    a_ijs = a_flat.reshape(E, H, 1)
    return h_is, a_ijs


# speedup vs baseline: 10.7628x; 1.0216x over previous
import functools

import jax
import jax.numpy as jnp
from jax import lax
from jax.experimental import pallas as pl
from jax.experimental.pallas import tpu as pltpu
from jax.experimental.pallas import tpu_sc as plsc

NC = 2
NS = 16
L = 16
NW = NC * NS


def _leaky(x):
    return jnp.where(x > 0, x, 0.2 * x)


def _pre_body(U, UW, H, node_ref, wcat_ref, bl_ref, ba_ref,
              wn_ref, pin_ref, pout_ref):
    x = node_ref[...]
    HU = H * U
    p = jnp.dot(x, wcat_ref[...], preferred_element_type=jnp.float32)
    wn = _leaky(p[:, :HU] + bl_ref[...])
    pin_ref[...] = p[:, HU:2 * HU] + ba_ref[...]
    pout_ref[...] = p[:, 2 * HU:]
    rb = x.shape[0]
    pad = (lax.broadcasted_iota(jnp.int32, (rb, UW - U), 1) == 0)
    for k in range(H):
        wn_ref[k, :, :U] = wn[:, k * U:(k + 1) * U]
        wn_ref[k, :, U:] = pad.astype(jnp.float32)


def _score_body(E, EW, BA, OB, H, U, UW,
                pin_hbm, pout_hbm, ii_hbm, io_hbm, al_hbm, a_hbm, e_hbm,
                ii_v, io_v, al_v, pin0, pout0, pin1, pout1, a_v, e_v,
                sg0, sg1):
    wid = lax.axis_index("s") * NC + lax.axis_index("c")
    base = wid * EW
    HU = H * U
    pltpu.sync_copy(al_hbm, al_v)
    pltpu.sync_copy(ii_hbm.at[pl.ds(base, EW)], ii_v)
    pltpu.sync_copy(io_hbm.at[pl.ds(base, EW)], io_v)
    NBLK = EW // BA
    NPAIR = NBLK // 2
    epg = L // H
    lanes = lax.iota(jnp.int32, L)
    tr_idx0 = (lanes % H) * (OB * BA) + lanes // H
    bufs = ((pin0, pout0, sg0), (pin1, pout1, sg1))
    GB = BA * HU * 4

    def start_gather(b, p):
        pin_b, pout_b, sg = bufs[p]
        pltpu.async_copy(pin_hbm.at[ii_v.at[pl.ds(b * BA, BA)]], pin_b, sg)
        pltpu.async_copy(pout_hbm.at[io_v.at[pl.ds(b * BA, BA)]], pout_b, sg)

    def wait_gather(p):
        pin_b, pout_b, sg = bufs[p]
        pltpu.make_async_copy(pin_hbm.at[pl.ds(0, BA)], pin_b, sg).wait()
        pltpu.make_async_copy(pin_hbm.at[pl.ds(0, BA)], pout_b, sg).wait()

    def compute_block(b, p):
        pin_b, pout_b, _ = bufs[p]
        q = b % OB

        def group_body(g, _):
            vec = jnp.zeros((L,), jnp.float32)
            for eo in range(epg):
                e = g * epg + eo
                for k in range(H):
                    acc = jnp.zeros((L,), jnp.float32)
                    for c in range(U // L):
                        sl = pl.ds(k * U + c * L, L)
                        x = pin_b[e, sl] + pout_b[e, sl]
                        acc = acc + _leaky(x) * al_v[k, pl.ds(c * L, L)]
                    vec = jnp.where(lanes == eo * H + k, jnp.sum(acc), vec)
            a_v[pl.ds(q * BA * H + g * L, L)] = vec
            plsc.store_scatter(e_v, [tr_idx0 + (q * BA + g * epg)],
                               jnp.exp(vec))
            return 0

        lax.fori_loop(0, BA * H // L, group_body, 0)

    start_gather(0, 0)
    start_gather(1, 1)

    def pair_body(bp, _):
        b0 = bp * 2
        for p in range(2):
            b = b0 + p
            wait_gather(p)
            compute_block(b, p)

            @pl.when(bp < NPAIR - 1)
            def _():
                start_gather(b + 2, p)

        @pl.when(bp % (OB // 2) == OB // 2 - 1)
        def _():
            grp0 = base + (b0 + 2 - OB) * BA
            pltpu.sync_copy(a_v, a_hbm.at[pl.ds(grp0 * H, OB * BA * H)])
            for k in range(H):
                pltpu.sync_copy(e_v.at[pl.ds(k * OB * BA, OB * BA)],
                                e_hbm.at[pl.ds(k * E + grp0, OB * BA)])
        return 0

    lax.fori_loop(0, NPAIR, pair_body, 0)


def _agg_body(E, N, Np, BB, NBC, H, HPC, U, UW,
              wn_hbm, ii_hbm, io_hbm, e_hbm, h_hbm,
              accum, ii_v, io_v, e_v, ib0, ib1, gi0, gi1, w0, w1,
              sg0, sg1, ss0, ss1):
    c = lax.axis_index("c")
    s = lax.axis_index("s")
    ES = E // NS
    RZ = Np // NS
    CH = NBC * BB
    SCB = BB * UW * 4
    GRP = BB // L
    CW = UW // L
    bufs = ((ib0, gi0, w0, sg0, ss0), (ib1, gi1, w1, sg1, ss1))

    for kl in range(HPC):
        head = c * HPC + kl

        def zw_body(i, _):
            w0[i // CW, pl.ds((i % CW) * L, L)] = jnp.zeros((L,), jnp.float32)
            return 0

        lax.fori_loop(0, BB * CW, zw_body, 0)

        def zc_body(i, _):
            pltpu.sync_copy(w0, accum.at[pl.ds(s * RZ + i * BB, BB)])
            return 0

        lax.fori_loop(0, RZ // BB, zc_body, 0)
        plsc.subcore_barrier()

        def build_and_gather(p, bo):
            ib, gi, w, sg, ss = bufs[p]
            for g in range(GRP):
                sl = pl.ds(g * L, L)
                ib[sl] = ii_v[pl.ds(bo + g * L, L)]
                gi[sl] = io_v[pl.ds(bo + g * L, L)] + head * N
            pltpu.async_copy(wn_hbm.at[gi], w, sg)

        def wait_gather(p):
            ib, gi, w, sg, ss = bufs[p]
            pltpu.make_async_copy(wn_hbm.at[pl.ds(0, BB)], w, sg).wait()

        def scale(p, bo):
            ib, gi, w, sg, ss = bufs[p]

            def scale_body(gg, _):
                g16 = e_v[pl.ds(bo + gg * L, L)]
                for eo in range(L):
                    for j in range(CW):
                        sl = pl.ds(j * L, L)
                        w[gg * L + eo, sl] = w[gg * L + eo, sl] * g16[eo]
                return 0

            lax.fori_loop(0, GRP, scale_body, 0)

        def scat(p):
            ib, gi, w, sg, ss = bufs[p]
            return pltpu.async_copy(w, accum.at[ib], ss, add=True)

        def chunk_body(chk, _):
            coff = s * ES + chk * CH
            pltpu.sync_copy(ii_hbm.at[pl.ds(coff, CH)], ii_v)
            pltpu.sync_copy(io_hbm.at[pl.ds(coff, CH)], io_v)
            pltpu.sync_copy(e_hbm.at[pl.ds(head * E + coff, CH)], e_v)
            build_and_gather(0, 0)
            build_and_gather(1, BB)

            def pair_body(bp, _):
                bo0 = bp * 2 * BB
                wait_gather(0)
                scale(0, bo0)
                sc0 = scat(0)
                wait_gather(1)
                scale(1, bo0 + BB)
                sc0.wait()
                build_and_gather(0, bo0 + 2 * BB)
                sc1 = scat(1)
                sc1.wait()
                build_and_gather(1, bo0 + 3 * BB)
                return 0

            lax.fori_loop(0, NBC // 2 - 1, pair_body, 0)
            bo0 = (NBC - 2) * BB
            wait_gather(0)
            scale(0, bo0)
            sc0 = scat(0)
            wait_gather(1)
            scale(1, bo0 + BB)
            sc0.wait()
            sc1 = scat(1)
            sc1.wait()
            return 0

        lax.fori_loop(0, ES // CH, chunk_body, 0)
        plsc.subcore_barrier()

        def wb_body(i, _):
            row0 = s * RZ + i * BB

            @pl.when(row0 + BB <= N)
            def _():
                pltpu.sync_copy(accum.at[pl.ds(row0, BB)], w0)

                def row_body(r, _):
                    dvec = w0[r, pl.ds(U, L)]
                    rvec = 1.0 / (dvec + 1e-16)
                    rec = rvec[0]
                    for j in range(U // L):
                        sl = pl.ds(j * L, L)
                        w0[r, sl] = _leaky(w0[r, sl] * rec)
                    return 0

                lax.fori_loop(0, BB, row_body, 0)
                pltpu.sync_copy(
                    w0.at[:, pl.ds(0, U)],
                    h_hbm.at[pl.ds(row0, BB), pl.ds(head * U, U)])
            return 0

        lax.fori_loop(0, RZ // BB, wb_body, 0)


def kernel(node, edge, edge_index, W_lin, b_lin, W_att, b_att, W_alpha):
    N, D = node.shape
    E = edge_index.shape[1]
    H, _, U = W_lin.shape
    HU = H * U
    UW = U + L
    HPC = H // NC

    w_cat = jnp.concatenate([
        W_lin.transpose(1, 0, 2).reshape(D, HU),
        W_att[:, :D, :].transpose(1, 0, 2).reshape(D, HU),
        W_att[:, D:, :].transpose(1, 0, 2).reshape(D, HU),
    ], axis=1)
    bl = b_lin.reshape(1, HU)
    ba = b_att.reshape(1, HU)
    alpha = W_alpha.reshape(H, U)
    idx_in = edge_index[0]
    idx_out = edge_index[1]

    RB = 1000
    grid = (N // RB,)
    wn, pin, pout = pl.pallas_call(
        functools.partial(_pre_body, U, UW, H),
        grid=grid,
        in_specs=[
            pl.BlockSpec((RB, D), lambda i: (i, 0)),
            pl.BlockSpec((D, 3 * HU), lambda i: (0, 0)),
            pl.BlockSpec((1, HU), lambda i: (0, 0)),
            pl.BlockSpec((1, HU), lambda i: (0, 0)),
        ],
        out_specs=[
            pl.BlockSpec((H, RB, UW), lambda i: (0, i, 0)),
            pl.BlockSpec((RB, HU), lambda i: (i, 0)),
            pl.BlockSpec((RB, HU), lambda i: (i, 0)),
        ],
        out_shape=[
            jax.ShapeDtypeStruct((H, N, UW), jnp.float32),
            jax.ShapeDtypeStruct((N, HU), jnp.float32),
            jax.ShapeDtypeStruct((N, HU), jnp.float32),
        ],
    )(node, w_cat, bl, ba)

    EW = E // NW
    BA = 40
    OB = 10
    mesh = plsc.VectorSubcoreMesh(core_axis_name="c", subcore_axis_name="s",
                                  num_cores=NC, num_subcores=NS)
    a_flat, exp_flat = pl.kernel(
        functools.partial(_score_body, E, EW, BA, OB, H, U, UW),
        out_type=[
            jax.ShapeDtypeStruct((E * H,), jnp.float32),
            jax.ShapeDtypeStruct((H * E,), jnp.float32),
        ],
        mesh=mesh,
        scratch_types=[
            pltpu.VMEM((EW,), jnp.int32),
            pltpu.VMEM((EW,), jnp.int32),
            pltpu.VMEM((H, U), jnp.float32),
            pltpu.VMEM((BA, HU), jnp.float32),
            pltpu.VMEM((BA, HU), jnp.float32),
            pltpu.VMEM((BA, HU), jnp.float32),
            pltpu.VMEM((BA, HU), jnp.float32),
            pltpu.VMEM((OB * BA * H,), jnp.float32),
            pltpu.VMEM((H * OB * BA,), jnp.float32),
            pltpu.SemaphoreType.DMA,
            pltpu.SemaphoreType.DMA,
        ],
        compiler_params=pltpu.CompilerParams(needs_layout_passes=False),
    )(pin, pout, idx_in, idx_out, alpha)

    BB = 80
    NBC = 10
    Np = -(-N // (64 * NS)) * (64 * NS)
    wn_flat = wn.reshape(H * N, UW)
    h_is = pl.kernel(
        functools.partial(_agg_body, E, N, Np, BB, NBC, H, HPC, U, UW),
        out_type=jax.ShapeDtypeStruct((N, HU), jnp.float32),
        mesh=mesh,
        scratch_types=[
            pltpu.VMEM_SHARED((Np, UW), jnp.float32),
            pltpu.VMEM((NBC * BB,), jnp.int32),
            pltpu.VMEM((NBC * BB,), jnp.int32),
            pltpu.VMEM((NBC * BB,), jnp.float32),
            pltpu.VMEM((BB,), jnp.int32),
            pltpu.VMEM((BB,), jnp.int32),
            pltpu.VMEM((BB,), jnp.int32),
            pltpu.VMEM((BB,), jnp.int32),
            pltpu.VMEM((BB, UW), jnp.float32),
            pltpu.VMEM((BB, UW), jnp.float32),
            pltpu.SemaphoreType.DMA,
            pltpu.SemaphoreType.DMA,
            pltpu.SemaphoreType.DMA,
            pltpu.SemaphoreType.DMA,
        ],
        compiler_params=pltpu.CompilerParams(
            use_tc_tiling_on_sc=False, needs_layout_passes=False),
    )(wn_flat, idx_in, idx_out, exp_flat)

    a_ijs = a_flat.reshape(E, H, 1)
    return h_is, a_ijs


# speedup vs baseline: 11.5014x; 1.0686x over previous
import functools

import jax
import jax.numpy as jnp
from jax import lax
from jax.experimental import pallas as pl
from jax.experimental.pallas import tpu as pltpu
from jax.experimental.pallas import tpu_sc as plsc

NC = 2
NS = 16
L = 16
NW = NC * NS


def _leaky(x):
    return jnp.where(x > 0, x, 0.2 * x)


def _pre_body(U, UW, H, node_ref, wcat_ref, bl_ref, ba_ref,
              wn_ref, pin_ref, pout_ref):
    x = node_ref[...]
    HU = H * U
    p = jnp.dot(x, wcat_ref[...], preferred_element_type=jnp.float32)
    wn = _leaky(p[:, :HU] + bl_ref[...])
    pin_ref[...] = p[:, HU:2 * HU] + ba_ref[...]
    pout_ref[...] = p[:, 2 * HU:]
    rb = x.shape[0]
    pad = (lax.broadcasted_iota(jnp.int32, (rb, UW - U), 1) == 0)
    for k in range(H):
        wn_ref[k, :, :U] = wn[:, k * U:(k + 1) * U]
        wn_ref[k, :, U:] = pad.astype(jnp.float32)


def _score_body(E, EW, BA, OB, H, U, UW,
                pin_hbm, pout_hbm, ii_hbm, io_hbm, al_hbm, a_hbm, e_hbm,
                ii_v, io_v, al_v, pin0, pout0, pin1, pout1, a_v, e_v,
                sg0, sg1):
    wid = lax.axis_index("s") * NC + lax.axis_index("c")
    base = wid * EW
    HU = H * U
    pltpu.sync_copy(al_hbm, al_v)
    pltpu.sync_copy(ii_hbm.at[pl.ds(base, EW)], ii_v)
    pltpu.sync_copy(io_hbm.at[pl.ds(base, EW)], io_v)
    NBLK = EW // BA
    NPAIR = NBLK // 2
    epg = L // H
    lanes = lax.iota(jnp.int32, L)
    tr_idx0 = (lanes % H) * (OB * BA) + lanes // H
    bufs = ((pin0, pout0, sg0), (pin1, pout1, sg1))
    GB = BA * HU * 4

    def start_gather(b, p):
        pin_b, pout_b, sg = bufs[p]
        pltpu.async_copy(pin_hbm.at[ii_v.at[pl.ds(b * BA, BA)]], pin_b, sg)
        pltpu.async_copy(pout_hbm.at[io_v.at[pl.ds(b * BA, BA)]], pout_b, sg)

    def wait_gather(p):
        pin_b, pout_b, sg = bufs[p]
        pltpu.make_async_copy(pin_hbm.at[pl.ds(0, BA)], pin_b, sg).wait()
        pltpu.make_async_copy(pin_hbm.at[pl.ds(0, BA)], pout_b, sg).wait()

    def compute_block(b, p):
        pin_b, pout_b, _ = bufs[p]
        q = b % OB

        def group_body(g, _):
            vec = jnp.zeros((L,), jnp.float32)
            for eo in range(epg):
                e = g * epg + eo
                for k in range(H):
                    acc = jnp.zeros((L,), jnp.float32)
                    for c in range(U // L):
                        sl = pl.ds(k * U + c * L, L)
                        x = pin_b[e, sl] + pout_b[e, sl]
                        acc = acc + _leaky(x) * al_v[k, pl.ds(c * L, L)]
                    vec = jnp.where(lanes == eo * H + k, jnp.sum(acc), vec)
            a_v[pl.ds(q * BA * H + g * L, L)] = vec
            plsc.store_scatter(e_v, [tr_idx0 + (q * BA + g * epg)],
                               jnp.exp(vec))
            return 0

        lax.fori_loop(0, BA * H // L, group_body, 0)

    start_gather(0, 0)
    start_gather(1, 1)

    def pair_body(bp, _):
        b0 = bp * 2
        for p in range(2):
            b = b0 + p
            wait_gather(p)
            compute_block(b, p)

            @pl.when(bp < NPAIR - 1)
            def _():
                start_gather(b + 2, p)

        @pl.when(bp % (OB // 2) == OB // 2 - 1)
        def _():
            grp0 = base + (b0 + 2 - OB) * BA
            pltpu.sync_copy(a_v, a_hbm.at[pl.ds(grp0 * H, OB * BA * H)])
            for k in range(H):
                pltpu.sync_copy(e_v.at[pl.ds(k * OB * BA, OB * BA)],
                                e_hbm.at[pl.ds(k * E + grp0, OB * BA)])
        return 0

    lax.fori_loop(0, NPAIR, pair_body, 0)


def _agg_body(E, N, Np, BB, NBC, H, HPC, U, UW,
              wn_hbm, ii_hbm, io_hbm, e_hbm, h_hbm,
              accum, ii_v, io_v, e_v, ib0, ib1, gi0, gi1, w0, w1,
              sg0, sg1, ss0, ss1):
    c = lax.axis_index("c")
    s = lax.axis_index("s")
    ES = E // NS
    RZ = Np // NS
    CH = NBC * BB
    SCB = BB * UW * 4
    GRP = BB // L
    CW = UW // L
    bufs = ((ib0, gi0, w0, sg0, ss0), (ib1, gi1, w1, sg1, ss1))

    for kl in range(HPC):
        head = c * HPC + kl

        def zw_body(i, _):
            w0[i // CW, pl.ds((i % CW) * L, L)] = jnp.zeros((L,), jnp.float32)
            return 0

        lax.fori_loop(0, BB * CW, zw_body, 0)

        def zc_body(i, _):
            pltpu.sync_copy(w0, accum.at[pl.ds(s * RZ + i * BB, BB)])
            return 0

        lax.fori_loop(0, RZ // BB, zc_body, 0)
        plsc.subcore_barrier()

        def build_and_gather(p, bo):
            ib, gi, w, sg, ss = bufs[p]
            for g in range(GRP):
                sl = pl.ds(g * L, L)
                ib[sl] = ii_v[pl.ds(bo + g * L, L)]
                gi[sl] = io_v[pl.ds(bo + g * L, L)] + head * N
            pltpu.async_copy(wn_hbm.at[gi], w, sg)

        def wait_gather(p):
            ib, gi, w, sg, ss = bufs[p]
            pltpu.make_async_copy(wn_hbm.at[pl.ds(0, BB)], w, sg).wait()

        def scale(p, bo):
            ib, gi, w, sg, ss = bufs[p]

            def scale_body(gg, _):
                g16 = e_v[pl.ds(bo + gg * L, L)]
                for eo in range(L):
                    for j in range(CW):
                        sl = pl.ds(j * L, L)
                        w[gg * L + eo, sl] = w[gg * L + eo, sl] * g16[eo]
                return 0

            lax.fori_loop(0, GRP, scale_body, 0)

        def scat(p):
            ib, gi, w, sg, ss = bufs[p]
            return pltpu.async_copy(w, accum.at[ib], ss, add=True)

        def chunk_body(chk, _):
            coff = s * ES + chk * CH
            pltpu.sync_copy(ii_hbm.at[pl.ds(coff, CH)], ii_v)
            pltpu.sync_copy(io_hbm.at[pl.ds(coff, CH)], io_v)
            pltpu.sync_copy(e_hbm.at[pl.ds(head * E + coff, CH)], e_v)
            build_and_gather(0, 0)
            build_and_gather(1, BB)

            def pair_body(bp, _):
                bo0 = bp * 2 * BB
                wait_gather(0)
                scale(0, bo0)
                sc0 = scat(0)
                wait_gather(1)
                scale(1, bo0 + BB)
                sc0.wait()
                build_and_gather(0, bo0 + 2 * BB)
                sc1 = scat(1)
                sc1.wait()
                build_and_gather(1, bo0 + 3 * BB)
                return 0

            lax.fori_loop(0, NBC // 2 - 1, pair_body, 0)
            bo0 = (NBC - 2) * BB
            wait_gather(0)
            scale(0, bo0)
            sc0 = scat(0)
            wait_gather(1)
            scale(1, bo0 + BB)
            sc0.wait()
            sc1 = scat(1)
            sc1.wait()
            return 0

        lax.fori_loop(0, ES // CH, chunk_body, 0)
        plsc.subcore_barrier()

        def wb_body(i, _):
            row0 = s * RZ + i * BB

            @pl.when(row0 + BB <= N)
            def _():
                pltpu.sync_copy(accum.at[pl.ds(row0, BB)], w0)

                def row_body(r, _):
                    dvec = w0[r, pl.ds(U, L)]
                    rvec = 1.0 / (dvec + 1e-16)
                    rec = rvec[0]
                    for j in range(U // L):
                        sl = pl.ds(j * L, L)
                        w0[r, sl] = _leaky(w0[r, sl] * rec)
                    return 0

                lax.fori_loop(0, BB, row_body, 0)
                pltpu.sync_copy(
                    w0.at[:, pl.ds(0, U)],
                    h_hbm.at[pl.ds(row0, BB), pl.ds(head * U, U)])
            return 0

        lax.fori_loop(0, RZ // BB, wb_body, 0)


def kernel(node, edge, edge_index, W_lin, b_lin, W_att, b_att, W_alpha):
    N, D = node.shape
    E = edge_index.shape[1]
    H, _, U = W_lin.shape
    HU = H * U
    UW = U + L
    HPC = H // NC

    w_cat = jnp.concatenate([
        W_lin.transpose(1, 0, 2).reshape(D, HU),
        W_att[:, :D, :].transpose(1, 0, 2).reshape(D, HU),
        W_att[:, D:, :].transpose(1, 0, 2).reshape(D, HU),
    ], axis=1)
    bl = b_lin.reshape(1, HU)
    ba = b_att.reshape(1, HU)
    alpha = W_alpha.reshape(H, U)
    idx_in = edge_index[0]
    idx_out = edge_index[1]

    RB = 1000
    grid = (N // RB,)
    wn, pin, pout = pl.pallas_call(
        functools.partial(_pre_body, U, UW, H),
        grid=grid,
        in_specs=[
            pl.BlockSpec((RB, D), lambda i: (i, 0)),
            pl.BlockSpec((D, 3 * HU), lambda i: (0, 0)),
            pl.BlockSpec((1, HU), lambda i: (0, 0)),
            pl.BlockSpec((1, HU), lambda i: (0, 0)),
        ],
        out_specs=[
            pl.BlockSpec((H, RB, UW), lambda i: (0, i, 0)),
            pl.BlockSpec((RB, HU), lambda i: (i, 0)),
            pl.BlockSpec((RB, HU), lambda i: (i, 0)),
        ],
        out_shape=[
            jax.ShapeDtypeStruct((H, N, UW), jnp.float32),
            jax.ShapeDtypeStruct((N, HU), jnp.float32),
            jax.ShapeDtypeStruct((N, HU), jnp.float32),
        ],
    )(node, w_cat, bl, ba)

    EW = E // NW
    BA = 40
    OB = 10
    mesh = plsc.VectorSubcoreMesh(core_axis_name="c", subcore_axis_name="s",
                                  num_cores=NC, num_subcores=NS)
    a_flat, exp_flat = pl.kernel(
        functools.partial(_score_body, E, EW, BA, OB, H, U, UW),
        out_type=[
            jax.ShapeDtypeStruct((E * H,), jnp.float32),
            jax.ShapeDtypeStruct((H * E,), jnp.float32),
        ],
        mesh=mesh,
        scratch_types=[
            pltpu.VMEM((EW,), jnp.int32),
            pltpu.VMEM((EW,), jnp.int32),
            pltpu.VMEM((H, U), jnp.float32),
            pltpu.VMEM((BA, HU), jnp.float32),
            pltpu.VMEM((BA, HU), jnp.float32),
            pltpu.VMEM((BA, HU), jnp.float32),
            pltpu.VMEM((BA, HU), jnp.float32),
            pltpu.VMEM((OB * BA * H,), jnp.float32),
            pltpu.VMEM((H * OB * BA,), jnp.float32),
            pltpu.SemaphoreType.DMA,
            pltpu.SemaphoreType.DMA,
        ],
        compiler_params=pltpu.CompilerParams(needs_layout_passes=False),
    )(pin, pout, idx_in, idx_out, alpha)

    BB = 80
    NBC = 50
    Np = -(-N // (64 * NS)) * (64 * NS)
    wn_flat = wn.reshape(H * N, UW)
    h_is = pl.kernel(
        functools.partial(_agg_body, E, N, Np, BB, NBC, H, HPC, U, UW),
        out_type=jax.ShapeDtypeStruct((N, HU), jnp.float32),
        mesh=mesh,
        scratch_types=[
            pltpu.VMEM_SHARED((Np, UW), jnp.float32),
            pltpu.VMEM((NBC * BB,), jnp.int32),
            pltpu.VMEM((NBC * BB,), jnp.int32),
            pltpu.VMEM((NBC * BB,), jnp.float32),
            pltpu.VMEM((BB,), jnp.int32),
            pltpu.VMEM((BB,), jnp.int32),
            pltpu.VMEM((BB,), jnp.int32),
            pltpu.VMEM((BB,), jnp.int32),
            pltpu.VMEM((BB, UW), jnp.float32),
            pltpu.VMEM((BB, UW), jnp.float32),
            pltpu.SemaphoreType.DMA,
            pltpu.SemaphoreType.DMA,
            pltpu.SemaphoreType.DMA,
            pltpu.SemaphoreType.DMA,
        ],
        compiler_params=pltpu.CompilerParams(
            use_tc_tiling_on_sc=False, needs_layout_passes=False),
    )(wn_flat, idx_in, idx_out, exp_flat)

    a_ijs = a_flat.reshape(E, H, 1)
    return h_is, a_ijs


# speedup vs baseline: 11.5255x; 1.0021x over previous
import functools

import jax
import jax.numpy as jnp
from jax import lax
from jax.experimental import pallas as pl
from jax.experimental.pallas import tpu as pltpu
from jax.experimental.pallas import tpu_sc as plsc

NC = 2
NS = 16
L = 16
NW = NC * NS


def _leaky(x):
    return jnp.where(x > 0, x, 0.2 * x)


def _pre_body(U, UW, H, node_ref, wcat_ref, bl_ref, ba_ref,
              wn_ref, pin_ref, pout_ref):
    x = node_ref[...]
    HU = H * U
    p = jnp.dot(x, wcat_ref[...], preferred_element_type=jnp.float32)
    wn = _leaky(p[:, :HU] + bl_ref[...])
    pin_ref[...] = p[:, HU:2 * HU] + ba_ref[...]
    pout_ref[...] = p[:, 2 * HU:]
    rb = x.shape[0]
    pad = (lax.broadcasted_iota(jnp.int32, (rb, UW - U), 1) == 0)
    for k in range(H):
        wn_ref[k, :, :U] = wn[:, k * U:(k + 1) * U]
        wn_ref[k, :, U:] = pad.astype(jnp.float32)


def _score_body(E, EW, BA, OB, H, U, UW,
                pin_hbm, pout_hbm, ii_hbm, io_hbm, al_hbm, a_hbm, e_hbm,
                ii_v, io_v, al_v, pin0, pout0, pin1, pout1, a_v, e_v,
                sg0, sg1):
    wid = lax.axis_index("s") * NC + lax.axis_index("c")
    base = wid * EW
    HU = H * U
    pltpu.sync_copy(al_hbm, al_v)
    pltpu.sync_copy(ii_hbm.at[pl.ds(base, EW)], ii_v)
    pltpu.sync_copy(io_hbm.at[pl.ds(base, EW)], io_v)
    NBLK = EW // BA
    NPAIR = NBLK // 2
    epg = L // H
    lanes = lax.iota(jnp.int32, L)
    tr_idx0 = (lanes % H) * (OB * BA) + lanes // H
    bufs = ((pin0, pout0, sg0), (pin1, pout1, sg1))
    GB = BA * HU * 4

    def start_gather(b, p):
        pin_b, pout_b, sg = bufs[p]
        pltpu.async_copy(pin_hbm.at[ii_v.at[pl.ds(b * BA, BA)]], pin_b, sg)
        pltpu.async_copy(pout_hbm.at[io_v.at[pl.ds(b * BA, BA)]], pout_b, sg)

    def wait_gather(p):
        pin_b, pout_b, sg = bufs[p]
        pltpu.make_async_copy(pin_hbm.at[pl.ds(0, BA)], pin_b, sg).wait()
        pltpu.make_async_copy(pin_hbm.at[pl.ds(0, BA)], pout_b, sg).wait()

    def compute_block(b, p):
        pin_b, pout_b, _ = bufs[p]
        q = b % OB

        def group_body(g, _):
            vec = jnp.zeros((L,), jnp.float32)
            for eo in range(epg):
                e = g * epg + eo
                for k in range(H):
                    acc = jnp.zeros((L,), jnp.float32)
                    for c in range(U // L):
                        sl = pl.ds(k * U + c * L, L)
                        x = pin_b[e, sl] + pout_b[e, sl]
                        acc = acc + _leaky(x) * al_v[k, pl.ds(c * L, L)]
                    vec = jnp.where(lanes == eo * H + k, jnp.sum(acc), vec)
            a_v[pl.ds(q * BA * H + g * L, L)] = vec
            plsc.store_scatter(e_v, [tr_idx0 + (q * BA + g * epg)],
                               jnp.exp(vec))
            return 0

        lax.fori_loop(0, BA * H // L, group_body, 0)

    start_gather(0, 0)
    start_gather(1, 1)

    def pair_body(bp, _):
        b0 = bp * 2
        for p in range(2):
            b = b0 + p
            wait_gather(p)
            compute_block(b, p)

            @pl.when(bp < NPAIR - 1)
            def _():
                start_gather(b + 2, p)

        @pl.when(bp % (OB // 2) == OB // 2 - 1)
        def _():
            grp0 = base + (b0 + 2 - OB) * BA
            pltpu.sync_copy(a_v, a_hbm.at[pl.ds(grp0 * H, OB * BA * H)])
            for k in range(H):
                pltpu.sync_copy(e_v.at[pl.ds(k * OB * BA, OB * BA)],
                                e_hbm.at[pl.ds(k * E + grp0, OB * BA)])
        return 0

    lax.fori_loop(0, NPAIR, pair_body, 0)


def _agg_body(E, N, Np, BB, NBC, H, HPC, U, UW,
              wn_hbm, ii_hbm, io_hbm, e_hbm, h_hbm,
              accum, ii_v, io_v, e_v, ib0, ib1, gi0, gi1, w0, w1,
              sg0, sg1, ss0, ss1):
    c = lax.axis_index("c")
    s = lax.axis_index("s")
    ES = E // NS
    RZ = Np // NS
    CH = NBC * BB
    SCB = BB * UW * 4
    GRP = BB // L
    CW = UW // L
    bufs = ((ib0, gi0, w0, sg0, ss0), (ib1, gi1, w1, sg1, ss1))

    for kl in range(HPC):
        head = c * HPC + kl

        def zw_body(i, _):
            w0[i // CW, pl.ds((i % CW) * L, L)] = jnp.zeros((L,), jnp.float32)
            return 0

        lax.fori_loop(0, BB * CW, zw_body, 0)

        def zc_body(i, _):
            pltpu.sync_copy(w0, accum.at[pl.ds(s * RZ + i * BB, BB)])
            return 0

        lax.fori_loop(0, RZ // BB, zc_body, 0)
        plsc.subcore_barrier()

        def build_and_gather(p, bo):
            ib, gi, w, sg, ss = bufs[p]
            for g in range(GRP):
                sl = pl.ds(g * L, L)
                ib[sl] = ii_v[pl.ds(bo + g * L, L)]
                gi[sl] = io_v[pl.ds(bo + g * L, L)] + head * N
            pltpu.async_copy(wn_hbm.at[gi], w, sg)

        def wait_gather(p):
            ib, gi, w, sg, ss = bufs[p]
            pltpu.make_async_copy(wn_hbm.at[pl.ds(0, BB)], w, sg).wait()

        def scale(p, bo):
            ib, gi, w, sg, ss = bufs[p]

            def scale_body(gg, _):
                g16 = e_v[pl.ds(bo + gg * L, L)]
                for eo in range(L):
                    for j in range(CW):
                        sl = pl.ds(j * L, L)
                        w[gg * L + eo, sl] = w[gg * L + eo, sl] * g16[eo]
                return 0

            lax.fori_loop(0, GRP, scale_body, 0)

        def scat(p):
            ib, gi, w, sg, ss = bufs[p]
            return pltpu.async_copy(w, accum.at[ib], ss, add=True)

        def chunk_body(chk, _):
            coff = s * ES + chk * CH
            pltpu.sync_copy(ii_hbm.at[pl.ds(coff, CH)], ii_v)
            pltpu.sync_copy(io_hbm.at[pl.ds(coff, CH)], io_v)
            pltpu.sync_copy(e_hbm.at[pl.ds(head * E + coff, CH)], e_v)
            build_and_gather(0, 0)
            build_and_gather(1, BB)

            def pair_body(bp, _):
                bo0 = bp * 2 * BB
                wait_gather(0)
                scale(0, bo0)
                sc0 = scat(0)
                wait_gather(1)
                scale(1, bo0 + BB)
                sc0.wait()
                build_and_gather(0, bo0 + 2 * BB)
                sc1 = scat(1)
                sc1.wait()
                build_and_gather(1, bo0 + 3 * BB)
                return 0

            lax.fori_loop(0, NBC // 2 - 1, pair_body, 0)
            bo0 = (NBC - 2) * BB
            wait_gather(0)
            scale(0, bo0)
            sc0 = scat(0)
            wait_gather(1)
            scale(1, bo0 + BB)
            sc0.wait()
            sc1 = scat(1)
            sc1.wait()
            return 0

        lax.fori_loop(0, ES // CH, chunk_body, 0)
        plsc.subcore_barrier()

        def wb_body(i, _):
            row0 = s * RZ + i * BB

            @pl.when(row0 + BB <= N)
            def _():
                pltpu.sync_copy(accum.at[pl.ds(row0, BB)], w0)

                def row_body(r, _):
                    dvec = w0[r, pl.ds(U, L)]
                    rvec = 1.0 / (dvec + 1e-16)
                    rec = rvec[0]
                    for j in range(U // L):
                        sl = pl.ds(j * L, L)
                        w0[r, sl] = _leaky(w0[r, sl] * rec)
                    return 0

                lax.fori_loop(0, BB, row_body, 0)
                pltpu.sync_copy(
                    w0.at[:, pl.ds(0, U)],
                    h_hbm.at[pl.ds(row0, BB), pl.ds(head * U, U)])
            return 0

        lax.fori_loop(0, RZ // BB, wb_body, 0)


def kernel(node, edge, edge_index, W_lin, b_lin, W_att, b_att, W_alpha):
    N, D = node.shape
    E = edge_index.shape[1]
    H, _, U = W_lin.shape
    HU = H * U
    UW = U + L
    HPC = H // NC

    w_cat = jnp.concatenate([
        W_lin.transpose(1, 0, 2).reshape(D, HU),
        W_att[:, :D, :].transpose(1, 0, 2).reshape(D, HU),
        W_att[:, D:, :].transpose(1, 0, 2).reshape(D, HU),
    ], axis=1)
    bl = b_lin.reshape(1, HU)
    ba = b_att.reshape(1, HU)
    alpha = W_alpha.reshape(H, U)
    idx_in = edge_index[0]
    idx_out = edge_index[1]

    RB = 1000
    grid = (N // RB,)
    wn, pin, pout = pl.pallas_call(
        functools.partial(_pre_body, U, UW, H),
        grid=grid,
        in_specs=[
            pl.BlockSpec((RB, D), lambda i: (i, 0)),
            pl.BlockSpec((D, 3 * HU), lambda i: (0, 0)),
            pl.BlockSpec((1, HU), lambda i: (0, 0)),
            pl.BlockSpec((1, HU), lambda i: (0, 0)),
        ],
        out_specs=[
            pl.BlockSpec((H, RB, UW), lambda i: (0, i, 0)),
            pl.BlockSpec((RB, HU), lambda i: (i, 0)),
            pl.BlockSpec((RB, HU), lambda i: (i, 0)),
        ],
        out_shape=[
            jax.ShapeDtypeStruct((H, N, UW), jnp.float32),
            jax.ShapeDtypeStruct((N, HU), jnp.float32),
            jax.ShapeDtypeStruct((N, HU), jnp.float32),
        ],
    )(node, w_cat, bl, ba)

    EW = E // NW
    BA = 40
    OB = 50
    mesh = plsc.VectorSubcoreMesh(core_axis_name="c", subcore_axis_name="s",
                                  num_cores=NC, num_subcores=NS)
    a_flat, exp_flat = pl.kernel(
        functools.partial(_score_body, E, EW, BA, OB, H, U, UW),
        out_type=[
            jax.ShapeDtypeStruct((E * H,), jnp.float32),
            jax.ShapeDtypeStruct((H * E,), jnp.float32),
        ],
        mesh=mesh,
        scratch_types=[
            pltpu.VMEM((EW,), jnp.int32),
            pltpu.VMEM((EW,), jnp.int32),
            pltpu.VMEM((H, U), jnp.float32),
            pltpu.VMEM((BA, HU), jnp.float32),
            pltpu.VMEM((BA, HU), jnp.float32),
            pltpu.VMEM((BA, HU), jnp.float32),
            pltpu.VMEM((BA, HU), jnp.float32),
            pltpu.VMEM((OB * BA * H,), jnp.float32),
            pltpu.VMEM((H * OB * BA,), jnp.float32),
            pltpu.SemaphoreType.DMA,
            pltpu.SemaphoreType.DMA,
        ],
        compiler_params=pltpu.CompilerParams(needs_layout_passes=False),
    )(pin, pout, idx_in, idx_out, alpha)

    BB = 80
    NBC = 50
    Np = -(-N // (64 * NS)) * (64 * NS)
    wn_flat = wn.reshape(H * N, UW)
    h_is = pl.kernel(
        functools.partial(_agg_body, E, N, Np, BB, NBC, H, HPC, U, UW),
        out_type=jax.ShapeDtypeStruct((N, HU), jnp.float32),
        mesh=mesh,
        scratch_types=[
            pltpu.VMEM_SHARED((Np, UW), jnp.float32),
            pltpu.VMEM((NBC * BB,), jnp.int32),
            pltpu.VMEM((NBC * BB,), jnp.int32),
            pltpu.VMEM((NBC * BB,), jnp.float32),
            pltpu.VMEM((BB,), jnp.int32),
            pltpu.VMEM((BB,), jnp.int32),
            pltpu.VMEM((BB,), jnp.int32),
            pltpu.VMEM((BB,), jnp.int32),
            pltpu.VMEM((BB, UW), jnp.float32),
            pltpu.VMEM((BB, UW), jnp.float32),
            pltpu.SemaphoreType.DMA,
            pltpu.SemaphoreType.DMA,
            pltpu.SemaphoreType.DMA,
            pltpu.SemaphoreType.DMA,
        ],
        compiler_params=pltpu.CompilerParams(
            use_tc_tiling_on_sc=False, needs_layout_passes=False),
    )(wn_flat, idx_in, idx_out, exp_flat)

    a_ijs = a_flat.reshape(E, H, 1)
    return h_is, a_ijs
